# baseline (speedup 1.0000x reference)
import functools

import jax
import jax.numpy as jnp
from jax import lax
from jax.experimental import pallas as pl
from jax.experimental.pallas import tpu as pltpu
from jax.experimental.pallas import tpu_sc as plsc

_VOCAB = 50265
_HIDDEN = 768
_MAX_POS = 512
_BATCH = 128
_SEQ = 512

_NC = 2
_NS = 16
_NW = _NC * _NS
_S_PER_W = _SEQ // _NW
_BB = 32
_NBB = _BATCH // _BB
_NSLOT = 2
_LANES = 16
_NSL = _HIDDEN // _LANES


def _emb_body(ids_t_hbm, words_hbm, pos_hbm, out_hbm,
              idx_all, p_v,
              g0, g1, o0, o1, oi0, oi1,
              sg0, sg1, so0, so1):
    wid = lax.axis_index("s") * _NC + lax.axis_index("c")
    g_bufs = (g0, g1)
    o_bufs = (o0, o1)
    oi_bufs = (oi0, oi1)
    g_sems = (sg0, sg1)
    o_sems = (so0, so1)

    pltpu.sync_copy(ids_t_hbm.at[pl.ds(wid * _S_PER_W * _BATCH,
                                       _S_PER_W * _BATCH)], idx_all)
    pltpu.sync_copy(pos_hbm.at[pl.ds(wid * _S_PER_W, _S_PER_W)], p_v)

    def _gather(sl, bbk, slot):
        idx = idx_all.at[pl.ds(sl * _BATCH + bbk * _BB, _BB)]
        return pltpu.make_async_copy(words_hbm.at[idx], g_bufs[slot],
                                     g_sems[slot])

    def _outwrite(slot):
        return pltpu.make_async_copy(o_bufs[slot],
                                     out_hbm.at[oi_bufs[slot]],
                                     o_sems[slot])

    iota16 = lax.iota(jnp.int32, _LANES)

    for k in range(_NSLOT):
        _gather(0, k, k).start()

    def pos_body(sl, _):
        s = wid * _S_PER_W + sl
        pvs = [p_v[sl, pl.ds(j * _LANES, _LANES)] for j in range(_NSL)]
        for bbk in range(_NBB):
            slot = bbk % _NSLOT
            _gather(sl, bbk, slot).wait()

            if bbk >= _NSLOT:
                _outwrite(slot).wait()
            else:
                @pl.when(sl > 0)
                def _():
                    _outwrite(slot).wait()

            @plsc.parallel_loop(0, _BB, 1, unroll=4)
            def _(i):
                for j in range(_NSL):
                    slc = pl.ds(j * _LANES, _LANES)
                    o_bufs[slot][i, slc] = g_bufs[slot][i, slc] + pvs[j]
            ovec = iota16 * _SEQ + (bbk * _BB * _SEQ + s)
            oi_bufs[slot][pl.ds(0, _LANES)] = ovec
            oi_bufs[slot][pl.ds(_LANES, _LANES)] = ovec + _LANES * _SEQ
            _outwrite(slot).start()

            if bbk < _NBB - _NSLOT:
                _gather(sl, bbk + _NSLOT, slot).start()
            else:
                @pl.when(sl < _S_PER_W - 1)
                def _():
                    _gather(sl + 1, bbk - (_NBB - _NSLOT), slot).start()

        return 0

    lax.fori_loop(0, _S_PER_W, pos_body, 0, unroll=False)
    for k in range(_NSLOT):
        _outwrite(k).wait()


_emb_kernel = functools.partial(
    pl.kernel,
    out_type=jax.ShapeDtypeStruct((_BATCH * _SEQ, _HIDDEN), jnp.float32),
    mesh=plsc.VectorSubcoreMesh(core_axis_name="c", subcore_axis_name="s"),
    scratch_types=[
        pltpu.VMEM((_S_PER_W * _BATCH,), jnp.int32),
        pltpu.VMEM((_S_PER_W, _HIDDEN), jnp.float32),
        pltpu.VMEM((_BB, _HIDDEN), jnp.float32),
        pltpu.VMEM((_BB, _HIDDEN), jnp.float32),
        pltpu.VMEM((_BB, _HIDDEN), jnp.float32),
        pltpu.VMEM((_BB, _HIDDEN), jnp.float32),
        pltpu.VMEM((_BB,), jnp.int32),
        pltpu.VMEM((_BB,), jnp.int32),
        pltpu.SemaphoreType.DMA,
        pltpu.SemaphoreType.DMA,
        pltpu.SemaphoreType.DMA,
        pltpu.SemaphoreType.DMA,
    ],
)(_emb_body)


def kernel(input_ids, word_embeddings, position_embeddings):
    ids_t = input_ids.T.reshape(-1)
    out = _emb_kernel(ids_t, word_embeddings, position_embeddings)
    return out.reshape(_BATCH, _SEQ, _HIDDEN)


# speedup vs baseline: 1.0068x; 1.0068x over previous
import functools

import jax
import jax.numpy as jnp
from jax import lax
from jax.experimental import pallas as pl
from jax.experimental.pallas import tpu as pltpu
from jax.experimental.pallas import tpu_sc as plsc

_VOCAB = 50265
_HIDDEN = 768
_MAX_POS = 512
_BATCH = 128
_SEQ = 512

_NC = 2
_NS = 16
_NW = _NC * _NS
_S_PER_W = _SEQ // _NW
_BB = 32
_NBB = _BATCH // _BB
_NSLOT = 2
_LANES = 16
_NSL = _HIDDEN // _LANES


def _emb_body(ids_t_hbm, words_hbm, pos_hbm, out_hbm,
              idx_all, p_v,
              g0, g1, o0, o1, oi0, oi1,
              sg0, sg1, so0, so1):
    wid = lax.axis_index("s") * _NC + lax.axis_index("c")
    g_bufs = (g0, g1)
    o_bufs = (o0, o1)
    oi_bufs = (oi0, oi1)
    g_sems = (sg0, sg1)
    o_sems = (so0, so1)

    pltpu.sync_copy(ids_t_hbm.at[pl.ds(wid * _S_PER_W * _BATCH,
                                       _S_PER_W * _BATCH)], idx_all)
    pltpu.sync_copy(pos_hbm.at[pl.ds(wid * _S_PER_W, _S_PER_W)], p_v)

    def _gather(sl, bbk, slot):
        idx = idx_all.at[pl.ds(sl * _BATCH + bbk * _BB, _BB)]
        return pltpu.make_async_copy(words_hbm.at[idx], g_bufs[slot],
                                     g_sems[slot])

    def _outwrite(slot):
        return pltpu.make_async_copy(o_bufs[slot],
                                     out_hbm.at[oi_bufs[slot]],
                                     o_sems[slot])

    iota16 = lax.iota(jnp.int32, _LANES)

    for k in range(_NSLOT):
        _gather(0, k, k).start()

    def pos_body(sl, _):
        s = wid * _S_PER_W + sl
        pvs = [p_v[sl, pl.ds(j * _LANES, _LANES)] for j in range(_NSL)]
        for bbk in range(_NBB):
            slot = bbk % _NSLOT
            _gather(sl, bbk, slot).wait()

            if bbk >= _NSLOT:
                _outwrite(slot).wait()
            else:
                @pl.when(sl > 0)
                def _():
                    _outwrite(slot).wait()

            @plsc.parallel_loop(0, _BB, 1, unroll=2)
            def _(i):
                for j in range(_NSL):
                    slc = pl.ds(j * _LANES, _LANES)
                    o_bufs[slot][i, slc] = g_bufs[slot][i, slc] + pvs[j]
            if bbk < _NBB - _NSLOT:
                _gather(sl, bbk + _NSLOT, slot).start()
            else:
                @pl.when(sl < _S_PER_W - 1)
                def _():
                    _gather(sl + 1, bbk - (_NBB - _NSLOT), slot).start()

            ovec = iota16 * _SEQ + (bbk * _BB * _SEQ + s)
            oi_bufs[slot][pl.ds(0, _LANES)] = ovec
            oi_bufs[slot][pl.ds(_LANES, _LANES)] = ovec + _LANES * _SEQ
            _outwrite(slot).start()

        return 0

    lax.fori_loop(0, _S_PER_W, pos_body, 0, unroll=False)
    for k in range(_NSLOT):
        _outwrite(k).wait()


_emb_kernel = functools.partial(
    pl.kernel,
    out_type=jax.ShapeDtypeStruct((_BATCH * _SEQ, _HIDDEN), jnp.float32),
    mesh=plsc.VectorSubcoreMesh(core_axis_name="c", subcore_axis_name="s"),
    scratch_types=[
        pltpu.VMEM((_S_PER_W * _BATCH,), jnp.int32),
        pltpu.VMEM((_S_PER_W, _HIDDEN), jnp.float32),
        pltpu.VMEM((_BB, _HIDDEN), jnp.float32),
        pltpu.VMEM((_BB, _HIDDEN), jnp.float32),
        pltpu.VMEM((_BB, _HIDDEN), jnp.float32),
        pltpu.VMEM((_BB, _HIDDEN), jnp.float32),
        pltpu.VMEM((_BB,), jnp.int32),
        pltpu.VMEM((_BB,), jnp.int32),
        pltpu.SemaphoreType.DMA,
        pltpu.SemaphoreType.DMA,
        pltpu.SemaphoreType.DMA,
        pltpu.SemaphoreType.DMA,
    ],
)(_emb_body)


def kernel(input_ids, word_embeddings, position_embeddings):
    ids_t = input_ids.T.reshape(-1)
    out = _emb_kernel(ids_t, word_embeddings, position_embeddings)
    return out.reshape(_BATCH, _SEQ, _HIDDEN)
